# baseline (speedup 1.0000x reference)
import functools

import jax
import jax.numpy as jnp
from jax import lax
from jax.experimental import pallas as pl
from jax.experimental.pallas import tpu as pltpu
from jax.experimental.pallas import tpu_sc as plsc

B = 16384
F = 26
V = 100000

NC = 2
NS = 16
NW = NC * NS

CHUNK = (B * F) // NW
SEG = 128
NSEG = CHUNK // SEG
PERIOD_VECS = 13
NPERIOD = CHUNK // (PERIOD_VECS * 16)
FIRE = 8
LANES = 16


def _make_kernel():
  mesh = plsc.VectorSubcoreMesh(core_axis_name="c", subcore_axis_name="s")

  @functools.partial(
      pl.kernel,
      mesh=mesh,
      out_type=jax.ShapeDtypeStruct((B * F,), jnp.float32),
      scratch_types=[
          pltpu.VMEM((CHUNK,), jnp.int32),
          pltpu.VMEM((CHUNK,), jnp.float32),
          pltpu.VMEM((PERIOD_VECS * LANES,), jnp.int32),
          pltpu.SemaphoreType.DMA,
      ],
  )
  def emb_gather(x_hbm, table_hbm, out_hbm, idx_v, vals_v, off_v, sem):
    wid = lax.axis_index("s") * NC + lax.axis_index("c")
    base = wid * CHUNK

    pltpu.sync_copy(x_hbm.at[pl.ds(base, CHUNK)], idx_v)

    lane = lax.iota(jnp.int32, LANES)
    for p in range(PERIOD_VECS):
      pos = lane + (p * LANES)
      off_v[pl.ds(p * LANES, LANES)] = (pos % F) * V

    def add_period(p, carry):
      j0 = p * (PERIOD_VECS * LANES)
      for q in range(PERIOD_VECS):
        sl = pl.ds(j0 + q * LANES, LANES)
        idx_v[sl] = idx_v[sl] + off_v[pl.ds(q * LANES, LANES)]
      return carry
    lax.fori_loop(0, NPERIOD, add_period, 0)

    def gather_group(g, carry):
      s0 = g * FIRE
      copies = []
      for t in range(FIRE):
        sl = pl.ds((s0 + t) * SEG, SEG)
        copies.append(
            pltpu.async_copy(table_hbm.at[idx_v.at[sl]], vals_v.at[sl], sem))
      for cp in copies:
        cp.wait()
      return carry
    lax.fori_loop(0, NSEG // FIRE, gather_group, 0)

    pltpu.sync_copy(vals_v, out_hbm.at[pl.ds(base, CHUNK)])

  return emb_gather


_EMB_GATHER = _make_kernel()


@jax.jit
def kernel(x, tables):
  xf = x.reshape(B * F)
  tf = tables.reshape(F * V)
  out = _EMB_GATHER(xf, tf)
  return out.reshape(B, F)
